# baseline (speedup 1.0000x reference)
import functools

import jax
import jax.numpy as jnp
from jax import lax
from jax.experimental import pallas as pl
from jax.experimental.pallas import tpu as pltpu
from jax.experimental.pallas import tpu_sc as plsc

NUM_ROWS = 1000
DIM = 64
BATCH = 4096 * 200

NC = 2
NS = 16
NW = NC * NS
B_PER_W = BATCH // NW
CHUNK = 128
SUPER = 512
N_SUPER = B_PER_W // SUPER


@functools.partial(
    pl.kernel,
    out_type=jax.ShapeDtypeStruct((BATCH, DIM), jnp.float32),
    mesh=plsc.VectorSubcoreMesh(core_axis_name="c", subcore_axis_name="s"),
    scratch_types=[
        pltpu.VMEM((B_PER_W,), jnp.int32),
        pltpu.VMEM((SUPER, DIM), jnp.float32),
        pltpu.VMEM((SUPER, DIM), jnp.float32),
        pltpu.SemaphoreType.DMA,
        pltpu.SemaphoreType.DMA,
        pltpu.SemaphoreType.DMA,
        pltpu.SemaphoreType.DMA,
    ],
    compiler_params=pltpu.CompilerParams(use_tc_tiling_on_sc=False),
)
def _lookup(x_hbm, table_hbm, out_hbm, idx_v, buf0, buf1, gsem0, gsem1,
            osem0, osem1):
    wid = lax.axis_index("s") * NC + lax.axis_index("c")
    base = wid * B_PER_W

    pltpu.sync_copy(x_hbm.at[pl.ds(base, B_PER_W)], idx_v)

    def fire_gathers(s, buf, gsem):
        for q in range(SUPER // CHUNK):
            src = table_hbm.at[idx_v.at[pl.ds(s * SUPER + q * CHUNK, CHUNK)]]
            pltpu.make_async_copy(src, buf.at[pl.ds(q * CHUNK, CHUNK)],
                                  gsem).start()

    def wait_gathers(buf, gsem):
        src = table_hbm.at[idx_v.at[pl.ds(0, SUPER)]]
        pltpu.make_async_copy(src, buf, gsem).wait()

    def start_out(s, buf, osem):
        pltpu.make_async_copy(buf, out_hbm.at[pl.ds(base + s * SUPER, SUPER)],
                              osem).start()

    def wait_out(buf, osem):
        pltpu.make_async_copy(buf, out_hbm.at[pl.ds(base, SUPER)],
                              osem).wait()

    @pl.loop(0, N_SUPER, step=2)
    def _(s):
        @pl.when(s >= 2)
        def _():
            wait_out(buf0, osem0)

        fire_gathers(s, buf0, gsem0)

        @pl.when(s >= 2)
        def _():
            wait_out(buf1, osem1)

        fire_gathers(s + 1, buf1, gsem1)

        wait_gathers(buf0, gsem0)
        start_out(s, buf0, osem0)
        wait_gathers(buf1, gsem1)
        start_out(s + 1, buf1, osem1)

    wait_out(buf0, osem0)
    wait_out(buf1, osem1)


def kernel(x, table):
    out = _lookup(x.reshape(-1), table)
    return out.reshape(x.shape[0], x.shape[1], DIM)


# speedup vs baseline: 1.3172x; 1.3172x over previous
import functools

import jax
import jax.numpy as jnp
from jax import lax
from jax.experimental import pallas as pl
from jax.experimental.pallas import tpu as pltpu
from jax.experimental.pallas import tpu_sc as plsc

NUM_ROWS = 1000
DIM = 64
BATCH = 4096 * 200

NC = 2
NS = 16
NW = NC * NS
B_PER_W = BATCH // NW
CHUNK = 128
SUPER = 512
N_SUPER = B_PER_W // SUPER


@functools.partial(
    pl.kernel,
    out_type=jax.ShapeDtypeStruct((BATCH, DIM), jnp.float32),
    mesh=plsc.VectorSubcoreMesh(core_axis_name="c", subcore_axis_name="s"),
    scratch_types=[
        pltpu.VMEM((B_PER_W,), jnp.int32),
        pltpu.VMEM((SUPER, DIM), jnp.float32),
        pltpu.VMEM((SUPER, DIM), jnp.float32),
        pltpu.VMEM_SHARED((NUM_ROWS, DIM), jnp.float32),
        pltpu.SemaphoreType.DMA,
        pltpu.SemaphoreType.DMA,
        pltpu.SemaphoreType.DMA,
        pltpu.SemaphoreType.DMA,
    ],
    compiler_params=pltpu.CompilerParams(use_tc_tiling_on_sc=False),
)
def _lookup(x_hbm, table_hbm, out_hbm, idx_v, buf0, buf1, table_sp,
            gsem0, gsem1, osem0, osem1):
    wid = lax.axis_index("s") * NC + lax.axis_index("c")
    base = wid * B_PER_W

    @pl.when(lax.axis_index("s") == 0)
    def _():
        pltpu.sync_copy(table_hbm, table_sp)

    plsc.subcore_barrier()

    pltpu.sync_copy(x_hbm.at[pl.ds(base, B_PER_W)], idx_v)

    def fire_gathers(s, buf, gsem):
        for q in range(SUPER // CHUNK):
            src = table_sp.at[idx_v.at[pl.ds(s * SUPER + q * CHUNK, CHUNK)]]
            pltpu.make_async_copy(src, buf.at[pl.ds(q * CHUNK, CHUNK)],
                                  gsem).start()

    def wait_gathers(buf, gsem):
        src = table_sp.at[idx_v.at[pl.ds(0, SUPER)]]
        pltpu.make_async_copy(src, buf, gsem).wait()

    def start_out(s, buf, osem):
        pltpu.make_async_copy(buf, out_hbm.at[pl.ds(base + s * SUPER, SUPER)],
                              osem).start()

    def wait_out(buf, osem):
        pltpu.make_async_copy(buf, out_hbm.at[pl.ds(base, SUPER)],
                              osem).wait()

    @pl.loop(0, N_SUPER, step=2)
    def _(s):
        @pl.when(s >= 2)
        def _():
            wait_out(buf0, osem0)

        fire_gathers(s, buf0, gsem0)

        @pl.when(s >= 2)
        def _():
            wait_out(buf1, osem1)

        fire_gathers(s + 1, buf1, gsem1)

        wait_gathers(buf0, gsem0)
        start_out(s, buf0, osem0)
        wait_gathers(buf1, gsem1)
        start_out(s + 1, buf1, osem1)

    wait_out(buf0, osem0)
    wait_out(buf1, osem1)


def kernel(x, table):
    out = _lookup(x.reshape(-1), table)
    return out.reshape(x.shape[0], x.shape[1], DIM)
